# baseline (speedup 1.0000x reference)
import functools

import jax
import jax.numpy as jnp
from jax import lax
from jax.experimental import pallas as pl
from jax.experimental.pallas import tpu as pltpu
from jax.experimental.pallas import tpu_sc as plsc

EMB = 64
BATCH = 16384
LANES = 128
NTILES = 7813
TPW = 246
DEPTH = 4
HCAP = 2080


@jax.jit
def _presuf_embed(pre, suf, W_pre, W_suf):
    info = plsc.get_sparse_core_info()
    nw = info.num_cores * info.num_subcores
    bpw = BATCH // nw

    WTp = W_pre.T
    WTs = W_suf.T

    mesh = plsc.VectorSubcoreMesh(core_axis_name="c", subcore_axis_name="s")

    scratch = (
        [pltpu.VMEM((EMB, LANES), jnp.float32) for _ in range(DEPTH)]
        + [pltpu.VMEM((HCAP,), jnp.int32), pltpu.VMEM((HCAP,), jnp.int32)]
        + [pltpu.VMEM((HCAP,), jnp.int32), pltpu.VMEM((HCAP,), jnp.int32)]
        + [pltpu.VMEM((BATCH // 4,), jnp.int32)]
        + [pltpu.VMEM((64, 2 * EMB), jnp.float32), pltpu.VMEM((64,), jnp.int32)]
        + [pltpu.SMEM((TPW + 1,), jnp.int32), pltpu.SMEM((TPW + 1,), jnp.int32)]
        + [pltpu.SemaphoreType.DMA for _ in range(DEPTH + 1)]
    )

    @functools.partial(
        pl.kernel,
        mesh=mesh,
        compiler_params=pltpu.CompilerParams(
            disable_bounds_checks=True, needs_layout_passes=False),
        out_type=(
            jax.ShapeDtypeStruct((BATCH + 1, 2 * EMB), jnp.float32),
            jax.ShapeDtypeStruct((BATCH + 1, 2 * EMB), jnp.float32),
        ),
        scratch_types=scratch,
    )
    def k1(pre_hbm, suf_hbm, wtp_hbm, wts_hbm, tmpp_hbm, tmps_hbm, *scr):
        rings = scr[0:DEPTH]
        hit_i, hit_b = scr[DEPTH], scr[DEPTH + 1]
        ghit_i, ghit_b = scr[DEPTH + 2], scr[DEPTH + 3]
        idxbuf_v = scr[DEPTH + 4]
        stage_v, srow_v = scr[DEPTH + 5], scr[DEPTH + 6]
        off_s, cur_s = scr[DEPTH + 7], scr[DEPTH + 8]
        sems = scr[DEPTH + 9:]
        ssem = sems[DEPTH]

        wid = lax.axis_index("s") * info.num_cores + lax.axis_index("c")
        lo_t = wid * TPW
        hi_t = jnp.minimum(lo_t + TPW, NTILES)
        ntc = hi_t - lo_t
        lo_v = lo_t * LANES
        hi_v = hi_t * LANES

        iot = lax.iota(jnp.int32, 16)
        lane0 = iot == 0

        for idx_hbm, wt_hbm, tmp_hbm, half in (
                (pre_hbm, wtp_hbm, tmpp_hbm, 0),
                (suf_hbm, wts_hbm, tmps_hbm, 1)):

            def fetch(slot, t):
                tv = pl.multiple_of((lo_t + t) * LANES, LANES)
                pltpu.async_copy(
                    wt_hbm.at[:, pl.ds(tv, LANES)], rings[slot], sems[slot])

            def waitslot(slot):
                pltpu.make_async_copy(
                    wt_hbm.at[:, pl.ds(0, LANES)], rings[slot],
                    sems[slot]).wait()

            cnt = jnp.int32(0)
            for q in range(4):
                pltpu.sync_copy(
                    idx_hbm.at[pl.ds(q * (BATCH // 4), BATCH // 4)],
                    idxbuf_v)

                def collect_q(u, c, q=q):
                    hv = idxbuf_v[pl.ds(u * 16, 16)]
                    m = (hv >= lo_v) & (hv < hi_v)
                    dst = plsc.cumsum(m.astype(jnp.int32)) - 1 + c
                    plsc.store_scatter(hit_i, [dst], hv, mask=m)
                    plsc.store_scatter(
                        hit_b, [dst], iot + (u * 16 + q * (BATCH // 4)),
                        mask=m)
                    return c + plsc.all_reduce_population_count(m)[0]

                cnt = lax.fori_loop(0, BATCH // 4 // 16, collect_q, cnt)

            def zero_cnt(t, _):
                off_s[t] = 0
                return 0
            lax.fori_loop(0, TPW + 1, zero_cnt, 0)

            def count(u, _):
                hv = hit_i[pl.ds(u * 16, 16)]
                val = (iot < (cnt - u * 16)).astype(jnp.int32)
                for il in range(16):
                    @pl.when(val[il] != 0)
                    def _(il=il, hv=hv):
                        t = (hv[il] >> 7) - lo_t
                        off_s[t + 1] = off_s[t + 1] + 1
                return 0
            lax.fori_loop(0, (cnt + 15) // 16, count, 0)

            def prefix(t, run):
                run2 = run + off_s[t + 1]
                off_s[t + 1] = run2
                cur_s[t] = run
                return run2
            lax.fori_loop(0, TPW, prefix, jnp.int32(0))

            def place(u, _):
                hv = hit_i[pl.ds(u * 16, 16)]
                bv = hit_b[pl.ds(u * 16, 16)]
                val = (iot < (cnt - u * 16)).astype(jnp.int32)
                for il in range(16):
                    @pl.when(val[il] != 0)
                    def _(il=il, hv=hv, bv=bv):
                        t = (hv[il] >> 7) - lo_t
                        o = cur_s[t]
                        cur_s[t] = o + 1
                        plsc.store_scatter(
                            ghit_i, [jnp.full((16,), o, jnp.int32)],
                            jnp.full((16,), hv[il], jnp.int32), mask=lane0)
                        plsc.store_scatter(
                            ghit_b, [jnp.full((16,), o, jnp.int32)],
                            jnp.full((16,), bv[il], jnp.int32), mask=lane0)
                return 0
            lax.fori_loop(0, (cnt + 15) // 16, place, 0)

            def reset_srow():
                for uu in range(4):
                    srow_v[pl.ds(uu * 16, 16)] = jnp.full(
                        (16,), BATCH, jnp.int32)

            reset_srow()

            for j in range(DEPTH):
                @pl.when(j < ntc)
                def _(j=j):
                    fetch(j, jnp.int32(j))

            def flush(st):
                pltpu.async_copy(stage_v, tmp_hbm.at[srow_v], ssem).wait()
                reset_srow()
                return st

            def do_tile(blk, st, j):
                t = blk * DEPTH + j

                def extract(h, st):
                    hv = ghit_i[pl.ds(h, 16)]
                    bv = ghit_b[pl.ds(h, 16)]
                    col = jnp.full((16,), hv[0] & (LANES - 1), jnp.int32)
                    r = st & 63
                    zero = jnp.zeros((16,), jnp.float32)
                    for kq in range(4):
                        fv = iot + 16 * kq
                        v = plsc.load_gather(rings[j], [fv, col])
                        stage_v[r, pl.ds(half * EMB + 16 * kq, 16)] = v
                        stage_v[r, pl.ds((1 - half) * EMB + 16 * kq, 16)] = zero
                    plsc.store_scatter(
                        srow_v, [jnp.full((16,), r, jnp.int32)],
                        jnp.full((16,), bv[0], jnp.int32), mask=lane0)
                    st = st + 1

                    @pl.when((st & 63) == 0)
                    def _():
                        flush(st)
                    return st

                waitslot(j)
                beg = off_s[t]
                end = off_s[t + 1]
                st = lax.fori_loop(beg, end, extract, st)
                tn = t + DEPTH

                @pl.when(tn < ntc)
                def _():
                    fetch(j, tn)
                return st

            def block(blk, st):
                for j in range(DEPTH):
                    st = lax.cond(
                        blk * DEPTH + j < ntc,
                        lambda st, blk=blk, j=j: do_tile(blk, st, j),
                        lambda st: st, st)
                return st

            nblk = (TPW + DEPTH - 1) // DEPTH
            st = lax.fori_loop(0, nblk, block, jnp.int32(0))
            flush(st)

    tmpp, tmps = k1(pre, suf, WTp, WTs)

    @functools.partial(
        pl.kernel,
        mesh=mesh,
        compiler_params=pltpu.CompilerParams(
            disable_bounds_checks=True, needs_layout_passes=False),
        out_type=jax.ShapeDtypeStruct((BATCH, 2 * EMB), jnp.float32),
        scratch_types=[
            pltpu.VMEM((128, 2 * EMB), jnp.float32),
            pltpu.VMEM((128, 2 * EMB), jnp.float32),
        ],
    )
    def k2(a_hbm, b_hbm, out_hbm, av, bv):
        wid = lax.axis_index("s") * info.num_cores + lax.axis_index("c")
        base = wid * bpw

        def chunk(c):
            cb = base + c * 128
            pltpu.sync_copy(a_hbm.at[pl.ds(cb, 128)], av)
            pltpu.sync_copy(b_hbm.at[pl.ds(cb, 128)], bv)

            def addrow(r):
                for u in range(8):
                    av[r, pl.ds(u * 16, 16)] = (
                        av[r, pl.ds(u * 16, 16)] + bv[r, pl.ds(u * 16, 16)])

            pl.loop(0, 128)(addrow)
            pltpu.sync_copy(av, out_hbm.at[pl.ds(cb, 128)])

        pl.loop(0, bpw // 128)(chunk)

    return k2(tmpp, tmps)


def kernel(unused, pre, suf, W_pre, W_suf):
    return _presuf_embed(pre, suf, W_pre, W_suf)


# speedup vs baseline: 1.8857x; 1.8857x over previous
import functools

import jax
import jax.numpy as jnp
from jax import lax
from jax.experimental import pallas as pl
from jax.experimental.pallas import tpu as pltpu
from jax.experimental.pallas import tpu_sc as plsc

EMB = 64
BATCH = 16384
LANES = 128
NTILES = 7813
TPW = 246
DEPTH = 4
HCAP = 2080


@jax.jit
def _presuf_embed(pre, suf, W_pre, W_suf):
    info = plsc.get_sparse_core_info()
    nw = info.num_cores * info.num_subcores
    bpw = BATCH // nw

    WTp = W_pre.T
    WTs = W_suf.T

    mesh = plsc.VectorSubcoreMesh(core_axis_name="c", subcore_axis_name="s")

    scratch = (
        [pltpu.VMEM((EMB, LANES), jnp.float32) for _ in range(DEPTH)]
        + [pltpu.VMEM((HCAP,), jnp.int32), pltpu.VMEM((HCAP,), jnp.int32)]
        + [pltpu.VMEM((HCAP,), jnp.int32), pltpu.VMEM((HCAP,), jnp.int32)]
        + [pltpu.VMEM((BATCH // 4,), jnp.int32)]
        + [pltpu.VMEM((16, 2 * EMB), jnp.float32), pltpu.VMEM((16,), jnp.int32)]
        + [pltpu.SMEM((TPW + 1,), jnp.int32), pltpu.SMEM((TPW + 1,), jnp.int32)]
        + [pltpu.SemaphoreType.DMA for _ in range(DEPTH + 1)]
    )

    @functools.partial(
        pl.kernel,
        mesh=mesh,
        compiler_params=pltpu.CompilerParams(
            disable_bounds_checks=True, needs_layout_passes=False),
        out_type=(
            jax.ShapeDtypeStruct((BATCH + 1, 2 * EMB), jnp.float32),
            jax.ShapeDtypeStruct((BATCH + 1, 2 * EMB), jnp.float32),
        ),
        scratch_types=scratch,
    )
    def k1(pre_hbm, suf_hbm, wtp_hbm, wts_hbm, tmpp_hbm, tmps_hbm, *scr):
        rings = scr[0:DEPTH]
        hit_i, hit_b = scr[DEPTH], scr[DEPTH + 1]
        ghit_i, ghit_b = scr[DEPTH + 2], scr[DEPTH + 3]
        idxbuf_v = scr[DEPTH + 4]
        stage_v, srow_v = scr[DEPTH + 5], scr[DEPTH + 6]
        off_s, cur_s = scr[DEPTH + 7], scr[DEPTH + 8]
        sems = scr[DEPTH + 9:]
        ssem = sems[DEPTH]

        wid = lax.axis_index("s") * info.num_cores + lax.axis_index("c")
        lo_t = wid * TPW
        hi_t = jnp.minimum(lo_t + TPW, NTILES)
        ntc = hi_t - lo_t
        lo_v = lo_t * LANES
        hi_v = hi_t * LANES

        iot = lax.iota(jnp.int32, 16)
        lane0 = iot == 0

        for idx_hbm, wt_hbm, tmp_hbm, half in (
                (pre_hbm, wtp_hbm, tmpp_hbm, 0),
                (suf_hbm, wts_hbm, tmps_hbm, 1)):

            def fetch(slot, t):
                tv = pl.multiple_of((lo_t + t) * LANES, LANES)
                pltpu.async_copy(
                    wt_hbm.at[:, pl.ds(tv, LANES)], rings[slot], sems[slot])

            def waitslot(slot):
                pltpu.make_async_copy(
                    wt_hbm.at[:, pl.ds(0, LANES)], rings[slot],
                    sems[slot]).wait()

            for j in range(DEPTH):
                @pl.when(j < ntc)
                def _(j=j):
                    fetch(j, jnp.int32(j))

            cnt = jnp.int32(0)
            for q in range(4):
                pltpu.sync_copy(
                    idx_hbm.at[pl.ds(q * (BATCH // 4), BATCH // 4)],
                    idxbuf_v)

                def collect_q(u, c, q=q):
                    hv = idxbuf_v[pl.ds(u * 16, 16)]
                    m = (hv >= lo_v) & (hv < hi_v)
                    dst = plsc.cumsum(m.astype(jnp.int32)) - 1 + c
                    plsc.store_scatter(hit_i, [dst], hv, mask=m)
                    plsc.store_scatter(
                        hit_b, [dst], iot + (u * 16 + q * (BATCH // 4)),
                        mask=m)
                    return c + plsc.all_reduce_population_count(m)[0]

                cnt = lax.fori_loop(0, BATCH // 4 // 16, collect_q, cnt)

            def zero_cnt(t, _):
                off_s[t] = 0
                return 0
            lax.fori_loop(0, TPW + 1, zero_cnt, 0)

            def count(u, _):
                hv = hit_i[pl.ds(u * 16, 16)]
                val = (iot < (cnt - u * 16)).astype(jnp.int32)
                for il in range(16):
                    @pl.when(val[il] != 0)
                    def _(il=il, hv=hv):
                        t = (hv[il] >> 7) - lo_t
                        off_s[t + 1] = off_s[t + 1] + 1
                return 0
            lax.fori_loop(0, (cnt + 15) // 16, count, 0)

            def prefix(t, run):
                run2 = run + off_s[t + 1]
                off_s[t + 1] = run2
                cur_s[t] = run
                return run2
            lax.fori_loop(0, TPW, prefix, jnp.int32(0))

            def place(u, _):
                hv = hit_i[pl.ds(u * 16, 16)]
                bv = hit_b[pl.ds(u * 16, 16)]
                val = (iot < (cnt - u * 16)).astype(jnp.int32)
                for il in range(16):
                    @pl.when(val[il] != 0)
                    def _(il=il, hv=hv, bv=bv):
                        t = (hv[il] >> 7) - lo_t
                        o = cur_s[t]
                        cur_s[t] = o + 1
                        plsc.store_scatter(
                            ghit_i, [jnp.full((16,), o, jnp.int32)],
                            jnp.full((16,), hv[il], jnp.int32), mask=lane0)
                        plsc.store_scatter(
                            ghit_b, [jnp.full((16,), o, jnp.int32)],
                            jnp.full((16,), bv[il], jnp.int32), mask=lane0)
                return 0
            lax.fori_loop(0, (cnt + 15) // 16, place, 0)

            def reset_srow():
                srow_v[pl.ds(0, 16)] = jnp.full((16,), BATCH, jnp.int32)

            reset_srow()

            def flush(st):
                pltpu.async_copy(stage_v, tmp_hbm.at[srow_v], ssem).wait()
                reset_srow()
                return st

            def do_tile(blk, st, j):
                t = blk * DEPTH + j

                def extract(h, st):
                    hv = ghit_i[pl.ds(h, 16)]
                    bv = ghit_b[pl.ds(h, 16)]
                    col = jnp.full((16,), hv[0] & (LANES - 1), jnp.int32)
                    r = st & 15
                    zero = jnp.zeros((16,), jnp.float32)
                    for kq in range(4):
                        fv = iot + 16 * kq
                        v = plsc.load_gather(rings[j], [fv, col])
                        stage_v[r, pl.ds(half * EMB + 16 * kq, 16)] = v
                        stage_v[r, pl.ds((1 - half) * EMB + 16 * kq, 16)] = zero
                    plsc.store_scatter(
                        srow_v, [jnp.full((16,), r, jnp.int32)],
                        jnp.full((16,), bv[0], jnp.int32), mask=lane0)
                    st = st + 1

                    @pl.when((st & 15) == 0)
                    def _():
                        flush(st)
                    return st

                waitslot(j)
                beg = off_s[t]
                end = off_s[t + 1]
                st = lax.fori_loop(beg, end, extract, st)
                tn = t + DEPTH

                @pl.when(tn < ntc)
                def _():
                    fetch(j, tn)
                return st

            def block(blk, st):
                for j in range(DEPTH):
                    st = lax.cond(
                        blk * DEPTH + j < ntc,
                        lambda st, blk=blk, j=j: do_tile(blk, st, j),
                        lambda st: st, st)
                return st

            nblk = (TPW + DEPTH - 1) // DEPTH
            st = lax.fori_loop(0, nblk, block, jnp.int32(0))
            flush(st)

    tmpp, tmps = k1(pre, suf, WTp, WTs)

    @functools.partial(
        pl.kernel,
        mesh=mesh,
        compiler_params=pltpu.CompilerParams(
            disable_bounds_checks=True, needs_layout_passes=False),
        out_type=jax.ShapeDtypeStruct((BATCH, 2 * EMB), jnp.float32),
        scratch_types=[
            pltpu.VMEM((128, 2 * EMB), jnp.float32),
            pltpu.VMEM((128, 2 * EMB), jnp.float32),
        ],
    )
    def k2(a_hbm, b_hbm, out_hbm, av, bv):
        wid = lax.axis_index("s") * info.num_cores + lax.axis_index("c")
        base = wid * bpw

        def chunk(c):
            cb = base + c * 128
            pltpu.sync_copy(a_hbm.at[pl.ds(cb, 128)], av)
            pltpu.sync_copy(b_hbm.at[pl.ds(cb, 128)], bv)

            def addrow(r):
                for u in range(8):
                    av[r, pl.ds(u * 16, 16)] = (
                        av[r, pl.ds(u * 16, 16)] + bv[r, pl.ds(u * 16, 16)])

            pl.loop(0, 128)(addrow)
            pltpu.sync_copy(av, out_hbm.at[pl.ds(cb, 128)])

        pl.loop(0, bpw // 128)(chunk)

    return k2(tmpp, tmps)


def kernel(unused, pre, suf, W_pre, W_suf):
    return _presuf_embed(pre, suf, W_pre, W_suf)


# speedup vs baseline: 1.9254x; 1.0211x over previous
import functools

import jax
import jax.numpy as jnp
from jax import lax
from jax.experimental import pallas as pl
from jax.experimental.pallas import tpu as pltpu
from jax.experimental.pallas import tpu_sc as plsc

EMB = 64
BATCH = 16384
LANES = 128
NTILES = 7813
TPW = 246
DEPTH = 4
HCAP = 2080


@jax.jit
def _presuf_embed(pre, suf, W_pre, W_suf):
    info = plsc.get_sparse_core_info()
    nw = info.num_cores * info.num_subcores
    bpw = BATCH // nw

    WTp = W_pre.T
    WTs = W_suf.T

    mesh = plsc.VectorSubcoreMesh(core_axis_name="c", subcore_axis_name="s")

    scratch = (
        [pltpu.VMEM((EMB, LANES), jnp.float32) for _ in range(DEPTH)]
        + [pltpu.VMEM((HCAP,), jnp.int32), pltpu.VMEM((HCAP,), jnp.int32)]
        + [pltpu.VMEM((HCAP,), jnp.int32), pltpu.VMEM((HCAP,), jnp.int32)]
        + [pltpu.VMEM((BATCH // 4,), jnp.int32)]
        + [pltpu.VMEM((16, 2 * EMB), jnp.float32), pltpu.VMEM((16, 2 * EMB), jnp.float32)]
        + [pltpu.VMEM((16,), jnp.int32), pltpu.VMEM((16,), jnp.int32)]
        + [pltpu.SMEM((TPW + 1,), jnp.int32), pltpu.SMEM((TPW + 1,), jnp.int32)]
        + [pltpu.SemaphoreType.DMA for _ in range(DEPTH + 2)]
    )

    @functools.partial(
        pl.kernel,
        mesh=mesh,
        compiler_params=pltpu.CompilerParams(
            disable_bounds_checks=True, needs_layout_passes=False),
        out_type=(
            jax.ShapeDtypeStruct((BATCH + 1, 2 * EMB), jnp.float32),
            jax.ShapeDtypeStruct((BATCH + 1, 2 * EMB), jnp.float32),
        ),
        scratch_types=scratch,
    )
    def k1(pre_hbm, suf_hbm, wtp_hbm, wts_hbm, tmpp_hbm, tmps_hbm, *scr):
        rings = scr[0:DEPTH]
        hit_i, hit_b = scr[DEPTH], scr[DEPTH + 1]
        ghit_i, ghit_b = scr[DEPTH + 2], scr[DEPTH + 3]
        idxbuf_v = scr[DEPTH + 4]
        stages = scr[DEPTH + 5:DEPTH + 7]
        srows = scr[DEPTH + 7:DEPTH + 9]
        off_s, cur_s = scr[DEPTH + 9], scr[DEPTH + 10]
        sems = scr[DEPTH + 11:]
        ssems = sems[DEPTH:DEPTH + 2]

        wid = lax.axis_index("s") * info.num_cores + lax.axis_index("c")
        lo_t = wid * TPW
        hi_t = jnp.minimum(lo_t + TPW, NTILES)
        ntc = hi_t - lo_t
        lo_v = lo_t * LANES
        hi_v = hi_t * LANES

        iot = lax.iota(jnp.int32, 16)
        lane0 = iot == 0

        for idx_hbm, wt_hbm, tmp_hbm, half in (
                (pre_hbm, wtp_hbm, tmpp_hbm, 0),
                (suf_hbm, wts_hbm, tmps_hbm, 1)):

            def fetch(slot, t):
                tv = pl.multiple_of((lo_t + t) * LANES, LANES)
                pltpu.async_copy(
                    wt_hbm.at[:, pl.ds(tv, LANES)], rings[slot], sems[slot])

            def waitslot(slot):
                pltpu.make_async_copy(
                    wt_hbm.at[:, pl.ds(0, LANES)], rings[slot],
                    sems[slot]).wait()

            for j in range(DEPTH):
                @pl.when(j < ntc)
                def _(j=j):
                    fetch(j, jnp.int32(j))

            cnt = jnp.int32(0)
            for q in range(4):
                pltpu.sync_copy(
                    idx_hbm.at[pl.ds(q * (BATCH // 4), BATCH // 4)],
                    idxbuf_v)

                def collect_q(u, c, q=q):
                    hv = idxbuf_v[pl.ds(u * 16, 16)]
                    m = (hv >= lo_v) & (hv < hi_v)
                    dst = plsc.cumsum(m.astype(jnp.int32)) - 1 + c
                    plsc.store_scatter(hit_i, [dst], hv, mask=m)
                    plsc.store_scatter(
                        hit_b, [dst], iot + (u * 16 + q * (BATCH // 4)),
                        mask=m)
                    return c + plsc.all_reduce_population_count(m)[0]

                cnt = lax.fori_loop(0, BATCH // 4 // 16, collect_q, cnt)

            def zero_cnt(t, _):
                off_s[t] = 0
                return 0
            lax.fori_loop(0, TPW + 1, zero_cnt, 0)

            def count(u, _):
                hv = hit_i[pl.ds(u * 16, 16)]
                val = (iot < (cnt - u * 16)).astype(jnp.int32)
                for il in range(16):
                    @pl.when(val[il] != 0)
                    def _(il=il, hv=hv):
                        t = (hv[il] >> 7) - lo_t
                        off_s[t + 1] = off_s[t + 1] + 1
                return 0
            lax.fori_loop(0, (cnt + 15) // 16, count, 0)

            def prefix(t, run):
                run2 = run + off_s[t + 1]
                off_s[t + 1] = run2
                cur_s[t] = run
                return run2
            lax.fori_loop(0, TPW, prefix, jnp.int32(0))

            def place(u, _):
                hv = hit_i[pl.ds(u * 16, 16)]
                bv = hit_b[pl.ds(u * 16, 16)]
                val = (iot < (cnt - u * 16)).astype(jnp.int32)
                for il in range(16):
                    @pl.when(val[il] != 0)
                    def _(il=il, hv=hv, bv=bv):
                        t = (hv[il] >> 7) - lo_t
                        o = cur_s[t]
                        cur_s[t] = o + 1
                        plsc.store_scatter(
                            ghit_i, [jnp.full((16,), o, jnp.int32)],
                            jnp.full((16,), hv[il], jnp.int32), mask=lane0)
                        plsc.store_scatter(
                            ghit_b, [jnp.full((16,), o, jnp.int32)],
                            jnp.full((16,), bv[il], jnp.int32), mask=lane0)
                return 0
            lax.fori_loop(0, (cnt + 15) // 16, place, 0)

            for gg in range(2):
                srows[gg][pl.ds(0, 16)] = jnp.full((16,), BATCH, jnp.int32)

            def issue_flush(gg):
                pltpu.async_copy(stages[gg], tmp_hbm.at[srows[gg]],
                                 ssems[gg])

            def wait_flush(gg):
                pltpu.make_async_copy(stages[gg], tmp_hbm.at[srows[gg]],
                                      ssems[gg]).wait()

            def do_tile(blk, st, j):
                t = blk * DEPTH + j

                def extract(h, st):
                    hv = ghit_i[pl.ds(h, 16)]
                    bv = ghit_b[pl.ds(h, 16)]
                    col = jnp.full((16,), hv[0] & (LANES - 1), jnp.int32)
                    r = st & 15
                    g = (st >> 4) & 1
                    zero = jnp.zeros((16,), jnp.float32)
                    vals = []
                    for kq in range(4):
                        fv = iot + 16 * kq
                        vals.append(plsc.load_gather(rings[j], [fv, col]))
                    for gg in range(2):
                        @pl.when(g == gg)
                        def _(gg=gg):
                            for kq in range(4):
                                stages[gg][
                                    r, pl.ds(half * EMB + 16 * kq, 16)] = (
                                        vals[kq])
                                stages[gg][
                                    r,
                                    pl.ds((1 - half) * EMB + 16 * kq, 16)
                                ] = zero
                            plsc.store_scatter(
                                srows[gg], [jnp.full((16,), r, jnp.int32)],
                                jnp.full((16,), bv[0], jnp.int32),
                                mask=lane0)
                    st = st + 1

                    @pl.when((st & 15) == 0)
                    def _():
                        gd = ((st >> 4) & 1) ^ 1
                        gn = (st >> 4) & 1
                        for gg in range(2):
                            @pl.when(gd == gg)
                            def _(gg=gg):
                                issue_flush(gg)

                            @pl.when((st >= 32) & (gn == gg))
                            def _(gg=gg):
                                wait_flush(gg)
                    return st

                waitslot(j)
                beg = off_s[t]
                end = off_s[t + 1]
                st = lax.fori_loop(beg, end, extract, st)
                tn = t + DEPTH

                @pl.when(tn < ntc)
                def _():
                    fetch(j, tn)
                return st

            def block(blk, st):
                for j in range(DEPTH):
                    st = lax.cond(
                        blk * DEPTH + j < ntc,
                        lambda st, blk=blk, j=j: do_tile(blk, st, j),
                        lambda st: st, st)
                return st

            nblk = (TPW + DEPTH - 1) // DEPTH
            st = lax.fori_loop(0, nblk, block, jnp.int32(0))
            @pl.when(st >= 16)
            def _():
                gd = ((st >> 4) & 1) ^ 1
                for gg in range(2):
                    @pl.when(gd == gg)
                    def _(gg=gg):
                        wait_flush(gg)
            for gg in range(2):
                issue_flush(gg)
                wait_flush(gg)

    tmpp, tmps = k1(pre, suf, WTp, WTs)

    @functools.partial(
        pl.kernel,
        mesh=mesh,
        compiler_params=pltpu.CompilerParams(
            disable_bounds_checks=True, needs_layout_passes=False),
        out_type=jax.ShapeDtypeStruct((BATCH, 2 * EMB), jnp.float32),
        scratch_types=[
            pltpu.VMEM((256, 2 * EMB), jnp.float32),
            pltpu.VMEM((256, 2 * EMB), jnp.float32),
        ],
    )
    def k2(a_hbm, b_hbm, out_hbm, av, bv):
        wid = lax.axis_index("s") * info.num_cores + lax.axis_index("c")
        base = wid * bpw

        def chunk(c):
            cb = base + c * 256
            pltpu.sync_copy(a_hbm.at[pl.ds(cb, 256)], av)
            pltpu.sync_copy(b_hbm.at[pl.ds(cb, 256)], bv)

            def addrow(r):
                for u in range(8):
                    av[r, pl.ds(u * 16, 16)] = (
                        av[r, pl.ds(u * 16, 16)] + bv[r, pl.ds(u * 16, 16)])

            pl.loop(0, 256)(addrow)
            pltpu.sync_copy(av, out_hbm.at[pl.ds(cb, 256)])

        pl.loop(0, bpw // 256)(chunk)

    return k2(tmpp, tmps)


def kernel(unused, pre, suf, W_pre, W_suf):
    return _presuf_embed(pre, suf, W_pre, W_suf)
